# baseline (speedup 1.0000x reference)
import functools

import jax
import jax.numpy as jnp
from jax import lax
from jax.experimental import pallas as pl
from jax.experimental.pallas import tpu as pltpu
from jax.experimental.pallas import tpu_sc as plsc

N_NODES = 10000
N_EDGES = 320000
DIM = 128
HDIM = DIM // 2
EDIM = 17

NC = 2
NS = 16
NW = NC * NS
EPW = N_EDGES // NW
CB = 40
NCHUNK = EPW // CB
LANES = 16
STRIP = 624
TAIL = N_NODES - NS * STRIP
LG = DIM // LANES
EPB = CB // 2 + 4


def _node_mm_body(x_ref, ws_ref, bs_ref, wd_ref, bd_ref, hs_ref, hd_ref):
    xb = x_ref[...]
    dn = (((1,), (1,)), ((), ()))
    hs_ref[...] = lax.dot_general(
        xb, ws_ref[...], dn, preferred_element_type=jnp.float32) + bs_ref[...]
    hd_ref[...] = lax.dot_general(
        xb, wd_ref[...], dn, preferred_element_type=jnp.float32) + bd_ref[...]


def _node_mm(x, W_src, b_src, W_dst, b_dst):
    nb = 10
    rb = N_NODES // nb
    return pl.pallas_call(
        _node_mm_body,
        grid=(nb,),
        in_specs=[
            pl.BlockSpec((rb, DIM), lambda i: (i, 0)),
            pl.BlockSpec((DIM, DIM), lambda i: (0, 0)),
            pl.BlockSpec((1, DIM), lambda i: (0, 0)),
            pl.BlockSpec((DIM, DIM), lambda i: (0, 0)),
            pl.BlockSpec((1, DIM), lambda i: (0, 0)),
        ],
        out_specs=[
            pl.BlockSpec((rb, DIM), lambda i: (i, 0)),
            pl.BlockSpec((rb, DIM), lambda i: (i, 0)),
        ],
        out_shape=[
            jax.ShapeDtypeStruct((N_NODES, DIM), jnp.float32),
            jax.ShapeDtypeStruct((N_NODES, DIM), jnp.float32),
        ],
    )(x, W_src, b_src.reshape(1, DIM), W_dst, b_dst.reshape(1, DIM))


def _edge_mm_body(eat_ref, we_ref, be_ref, ep_ref):
    eat = eat_ref[...]
    dn = (((0,), (1,)), ((), ()))
    y = lax.dot_general(
        eat, we_ref[...], dn,
        preferred_element_type=jnp.float32) + be_ref[...]
    eb = y.shape[0]
    yb = lax.bitcast_convert_type(
        y.astype(jnp.bfloat16), jnp.uint16).astype(jnp.uint32)
    yr = yb.reshape(eb // 2, 2, DIM)
    w = yr[:, 0, :] | (yr[:, 1, :] << 16)
    ep_ref[...] = lax.bitcast_convert_type(w, jnp.int32)


def _edge_mm(ea_t, W_edge, b_edge):
    nb = 50
    eb = N_EDGES // nb
    return pl.pallas_call(
        _edge_mm_body,
        grid=(nb,),
        in_specs=[pl.BlockSpec((EDIM, eb), lambda i: (0, i)),
                  pl.BlockSpec((DIM, EDIM), lambda i: (0, 0)),
                  pl.BlockSpec((1, DIM), lambda i: (0, 0))],
        out_specs=pl.BlockSpec((eb // 2, DIM), lambda i: (i, 0)),
        out_shape=jax.ShapeDtypeStruct((N_EDGES // 2, DIM), jnp.int32),
    )(ea_t, W_edge, b_edge.reshape(1, DIM))


_SC_MESH = plsc.VectorSubcoreMesh(
    core_axis_name="c", subcore_axis_name="s", num_cores=NC, num_subcores=NS)


@functools.partial(
    pl.kernel,
    out_type=jax.ShapeDtypeStruct((NC, N_NODES, DIM), jnp.float32),
    mesh=_SC_MESH,
    compiler_params=pltpu.CompilerParams(needs_layout_passes=False),
    scratch_types=[
        pltpu.VMEM((CB,), jnp.int32),
        pltpu.VMEM((CB,), jnp.int32),
        pltpu.VMEM((CB,), jnp.int32),
        pltpu.VMEM((CB,), jnp.int32),
        pltpu.VMEM((CB,), jnp.int32),
        pltpu.VMEM((CB,), jnp.int32),
        pltpu.VMEM((CB, DIM), jnp.float32),
        pltpu.VMEM((CB, DIM), jnp.float32),
        pltpu.VMEM((CB, DIM), jnp.float32),
        pltpu.VMEM((CB, DIM), jnp.float32),
        pltpu.VMEM((EPB, DIM), jnp.int32),
        pltpu.VMEM((EPB, DIM), jnp.int32),
        pltpu.VMEM((CB, DIM), jnp.float32),
        pltpu.VMEM((CB, DIM), jnp.float32),
        pltpu.VMEM((LANES,), jnp.float32),
        pltpu.VMEM_SHARED((N_NODES, DIM), jnp.float32),
        pltpu.SemaphoreType.DMA,
        pltpu.SemaphoreType.DMA,
        pltpu.SemaphoreType.DMA,
        pltpu.SemaphoreType.DMA,
        pltpu.SemaphoreType.DMA,
        pltpu.SemaphoreType.DMA,
        pltpu.SemaphoreType.DMA,
        pltpu.SemaphoreType.DMA,
    ],
)
def _edge_scatter(hs_hbm, hd_hbm, ep_hbm, src_hbm, dst_hbm, a_hbm, out_hbm,
                  si0, si1, dg0, dg1, ds0, ds1,
                  ba0, ba1, bb0, bb1, bc0, bc1, bz0, bz1,
                  a_v, acc, semg0, semg1, semsc0, semsc1,
                  semi0, semi1, semd0, semd1):
    c = lax.axis_index("c")
    s = lax.axis_index("s")
    wid = c * NS + s
    sibuf = (si0, si1)
    dgbuf = (dg0, dg1)
    dsbuf = (ds0, ds1)
    buf_a = (ba0, ba1)
    buf_b = (bb0, bb1)
    buf_c = (bc0, bc1)
    buf_z = (bz0, bz1)
    semg = (semg0, semg1)
    semsc = (semsc0, semsc1)
    semi = (semi0, semi1)
    semd = (semd0, semd1)

    zero = jnp.zeros((LANES,), jnp.float32)

    def _zrow(e, _):
        for j in range(DIM // LANES):
            bz0[e, pl.ds(j * LANES, LANES)] = zero
        return 0

    lax.fori_loop(0, CB, _zrow, 0)
    base_row = s * STRIP
    for k in range(STRIP // CB):
        pltpu.sync_copy(bz0, acc.at[pl.ds(base_row + k * CB, CB)])
    rem = STRIP % CB
    if rem:
        pltpu.sync_copy(bz0.at[pl.ds(0, rem)],
                        acc.at[pl.ds(base_row + (STRIP // CB) * CB, rem)])

    @pl.when(s == NS - 1)
    def _():
        pltpu.sync_copy(bz0.at[pl.ds(0, TAIL)],
                        acc.at[pl.ds(NS * STRIP, TAIL)])

    pltpu.sync_copy(a_hbm, a_v)
    a = a_v[...]
    himask = jnp.full((LANES,), -65536, jnp.int32)
    plsc.subcore_barrier()

    ebase = wid * EPW
    epbase = wid * (EPW // 2)

    def _issue_idx(ci, r):
        isl = pl.ds(ebase + ci * CB, CB)
        pltpu.async_copy(src_hbm.at[isl], sibuf[r], semi[r])
        pltpu.async_copy(dst_hbm.at[isl], dgbuf[r], semi[r])

    def _wait_idx(r):
        pltpu.make_async_copy(src_hbm.at[pl.ds(0, CB)], sibuf[r],
                              semi[r]).wait()
        pltpu.make_async_copy(dst_hbm.at[pl.ds(0, CB)], dgbuf[r],
                              semi[r]).wait()

    def _issue_dss(ci, r):
        pltpu.async_copy(dst_hbm.at[pl.ds(ebase + ci * CB, CB)], dsbuf[r],
                         semd[r])

    def _wait_dss(r):
        pltpu.make_async_copy(dst_hbm.at[pl.ds(0, CB)], dsbuf[r],
                              semd[r]).wait()

    def _issue_gathers(ci, r):
        pltpu.async_copy(hs_hbm.at[sibuf[r]], buf_a[r], semg[r])
        pltpu.async_copy(hd_hbm.at[dgbuf[r]], buf_b[r], semg[r])
        ep_start = pl.multiple_of(epbase + ci * (CB // 2) - 4 * r, 8)
        pltpu.async_copy(ep_hbm.at[pl.ds(ep_start, EPB)], buf_c[r], semg[r])

    def _wait_gathers(r):
        pltpu.make_async_copy(hs_hbm.at[sibuf[r]], buf_a[r], semg[r]).wait()
        pltpu.make_async_copy(hd_hbm.at[dgbuf[r]], buf_b[r], semg[r]).wait()
        pltpu.make_async_copy(ep_hbm.at[pl.ds(0, EPB)], buf_c[r],
                              semg[r]).wait()

    def _compute(r):
        ra, rb, rc, rz = buf_a[r], buf_b[r], buf_c[r], buf_z[r]
        pad = 4 * r

        def _epair(u, _):
            e0 = 2 * u
            e1 = 2 * u + 1
            for g in range(LG):
                sl = pl.ds(g * LANES, LANES)
                wv = rc[pad + u, sl]
                c0 = plsc.bitcast(lax.shift_left(wv, 16), jnp.float32)
                c1 = plsc.bitcast(lax.bitwise_and(wv, himask), jnp.float32)
                v0 = ra[e0, sl] + rb[e0, sl] + c0
                v1 = ra[e1, sl] + rb[e1, sl] + c1
                rz[e0, sl] = (jnp.maximum(v0, 0.0)
                              + a * jnp.minimum(v0, 0.0))
                rz[e1, sl] = (jnp.maximum(v1, 0.0)
                              + a * jnp.minimum(v1, 0.0))
            return 0

        lax.fori_loop(0, CB // 2, _epair, 0)

    def _issue_scatter(r):
        pltpu.async_copy(buf_z[r], acc.at[dsbuf[r]], semsc[r], add=True)

    def _wait_scatter(r):
        pltpu.make_async_copy(buf_z[r], acc.at[dsbuf[r]], semsc[r]).wait()

    _issue_idx(0, 0)
    _wait_idx(0)
    _issue_gathers(0, 0)
    _issue_idx(1, 1)
    _wait_idx(1)
    _issue_gathers(1, 1)
    _wait_gathers(0)
    _issue_idx(2, 0)
    _issue_dss(0, 0)
    _compute(0)
    _wait_dss(0)
    _issue_scatter(0)
    _wait_idx(0)
    _issue_gathers(2, 0)
    _wait_gathers(1)
    _issue_idx(3, 1)
    _issue_dss(1, 1)
    _compute(1)
    _wait_dss(1)
    _issue_scatter(1)

    def _half(ci, p):
        q = 1 - p
        _wait_idx(q)
        _issue_gathers(ci + 1, q)
        _wait_gathers(p)
        _issue_idx(ci + 2, p)
        _wait_scatter(p)
        _issue_dss(ci, p)
        _compute(p)
        _wait_dss(p)
        _issue_scatter(p)

    def _body(k, _):
        _half(2 * k, 0)
        _half(2 * k + 1, 1)
        return 0

    lax.fori_loop(1, NCHUNK // 2 - 1, _body, 0)

    _wait_idx(1)
    _issue_gathers(NCHUNK - 1, 1)
    _wait_gathers(0)
    _wait_scatter(0)
    _issue_dss(NCHUNK - 2, 0)
    _compute(0)
    _wait_dss(0)
    _issue_scatter(0)
    _wait_gathers(1)
    _wait_scatter(1)
    _issue_dss(NCHUNK - 1, 1)
    _compute(1)
    _wait_dss(1)
    _issue_scatter(1)
    _wait_scatter(0)
    _wait_scatter(1)

    plsc.subcore_barrier()
    pltpu.sync_copy(acc.at[pl.ds(base_row, STRIP)],
                    out_hbm.at[c].at[pl.ds(base_row, STRIP)])

    @pl.when(s == NS - 1)
    def _():
        pltpu.sync_copy(acc.at[pl.ds(NS * STRIP, TAIL)],
                        out_hbm.at[c].at[pl.ds(NS * STRIP, TAIL)])


def _bn_stats_body(p_ref, sum_ref, sq_ref):
    i = pl.program_id(0)
    h = p_ref[0] + p_ref[1]
    s = jnp.sum(h, axis=0, keepdims=True)
    q = jnp.sum(h * h, axis=0, keepdims=True)

    @pl.when(i == 0)
    def _():
        sum_ref[...] = s
        sq_ref[...] = q

    @pl.when(i > 0)
    def _():
        sum_ref[...] += s
        sq_ref[...] += q


def _bn_stats(partial):
    nb = 10
    rb = N_NODES // nb
    return pl.pallas_call(
        _bn_stats_body,
        grid=(nb,),
        in_specs=[pl.BlockSpec((NC, rb, DIM), lambda i: (0, i, 0))],
        out_specs=[
            pl.BlockSpec((1, DIM), lambda i: (0, 0)),
            pl.BlockSpec((1, DIM), lambda i: (0, 0)),
        ],
        out_shape=[
            jax.ShapeDtypeStruct((1, DIM), jnp.float32),
            jax.ShapeDtypeStruct((1, DIM), jnp.float32),
        ],
    )(partial)


def _bn_apply_body(p_ref, sum_ref, sq_ref, g_ref, b_ref, a_ref, out_ref):
    inv_n = 1.0 / N_NODES
    mean = sum_ref[...] * inv_n
    var = sq_ref[...] * inv_n - mean * mean
    inv = lax.rsqrt(var + 1e-5)
    scale = inv * g_ref[...]
    shift = b_ref[...] - mean * scale
    h = p_ref[0] + p_ref[1]
    y = h * scale + shift
    a = a_ref[0]
    out_ref[...] = jnp.maximum(y, 0.0) + a * jnp.minimum(y, 0.0)


def _bn_apply(partial, sums, sqs, gamma, beta, a_arr):
    nb = 10
    rb = N_NODES // nb
    return pl.pallas_call(
        _bn_apply_body,
        grid=(nb,),
        in_specs=[
            pl.BlockSpec((NC, rb, DIM), lambda i: (0, i, 0)),
            pl.BlockSpec((1, DIM), lambda i: (0, 0)),
            pl.BlockSpec((1, DIM), lambda i: (0, 0)),
            pl.BlockSpec((1, DIM), lambda i: (0, 0)),
            pl.BlockSpec((1, DIM), lambda i: (0, 0)),
            pl.BlockSpec(memory_space=pltpu.SMEM),
        ],
        out_specs=pl.BlockSpec((rb, DIM), lambda i: (i, 0)),
        out_shape=jax.ShapeDtypeStruct((N_NODES, DIM), jnp.float32),
    )(partial, sums, sqs, gamma.reshape(1, DIM), beta.reshape(1, DIM), a_arr)


def kernel(x, edge_index, edge_attr, W_src, b_src, W_dst, b_dst,
           W_edge, b_edge, prelu_a, bn_gamma, bn_beta):
    hs, hd = _node_mm(x, W_src, b_src, W_dst, b_dst)
    ep = _edge_mm(edge_attr.T, W_edge, b_edge)
    src = edge_index[0]
    dst = edge_index[1]
    a_vec = jnp.full((LANES,), prelu_a, jnp.float32)
    partial = _edge_scatter(hs, hd, ep, src, dst, a_vec)
    sums, sqs = _bn_stats(partial)
    a_arr = jnp.full((1,), prelu_a, jnp.float32)
    return _bn_apply(partial, sums, sqs, bn_gamma, bn_beta, a_arr)


# speedup vs baseline: 1.4688x; 1.4688x over previous
import functools

import jax
import jax.numpy as jnp
from jax import lax
from jax.experimental import pallas as pl
from jax.experimental.pallas import tpu as pltpu
from jax.experimental.pallas import tpu_sc as plsc

N_NODES = 10000
N_EDGES = 320000
DIM = 128
HDIM = DIM // 2
EDIM = 17

NC = 2
NS = 16
NW = NC * NS
EPW = N_EDGES // NW
CB = 40
NCHUNK = EPW // CB
LANES = 16
STRIP = 624
TAIL = N_NODES - NS * STRIP
GROUPS = DIM // 32

_IDX_A = tuple(g * 32 + w for g in range(GROUPS) for w in range(LANES))
_IDX_B = tuple(i + LANES for i in _IDX_A)


def _node_mm_body(x_ref, ws_ref, bs_ref, wd_ref, bd_ref, hs_ref, hd_ref):
    xb = x_ref[...]
    dn = (((1,), (1,)), ((), ()))
    hs_ref[...] = lax.dot_general(
        xb, ws_ref[...], dn, preferred_element_type=jnp.float32) + bs_ref[...]
    hd_ref[...] = lax.dot_general(
        xb, wd_ref[...], dn, preferred_element_type=jnp.float32) + bd_ref[...]


def _node_mm(x, W_src, b_src, W_dst, b_dst):
    nb = 10
    rb = N_NODES // nb
    return pl.pallas_call(
        _node_mm_body,
        grid=(nb,),
        in_specs=[
            pl.BlockSpec((rb, DIM), lambda i: (i, 0)),
            pl.BlockSpec((DIM, DIM), lambda i: (0, 0)),
            pl.BlockSpec((1, DIM), lambda i: (0, 0)),
            pl.BlockSpec((DIM, DIM), lambda i: (0, 0)),
            pl.BlockSpec((1, DIM), lambda i: (0, 0)),
        ],
        out_specs=[
            pl.BlockSpec((rb, DIM), lambda i: (i, 0)),
            pl.BlockSpec((rb, DIM), lambda i: (i, 0)),
        ],
        out_shape=[
            jax.ShapeDtypeStruct((N_NODES, DIM), jnp.float32),
            jax.ShapeDtypeStruct((N_NODES, DIM), jnp.float32),
        ],
    )(x, W_src, b_src.reshape(1, DIM), W_dst, b_dst.reshape(1, DIM))


def _edge_mm_body(eat_ref, wea_ref, bea_ref, web_ref, beb_ref, ep_ref):
    eat = eat_ref[...]
    dn = (((0,), (1,)), ((), ()))

    def mm(w_ref, b_ref):
        return lax.dot_general(
            eat, w_ref[...], dn,
            preferred_element_type=jnp.float32) + b_ref[...]

    ya = mm(wea_ref, bea_ref)
    yb = mm(web_ref, beb_ref)
    la = lax.bitcast_convert_type(ya.astype(jnp.bfloat16), jnp.uint16)
    lb = lax.bitcast_convert_type(yb.astype(jnp.bfloat16), jnp.uint16)
    w = la.astype(jnp.uint32) | (lb.astype(jnp.uint32) << 16)
    ep_ref[...] = lax.bitcast_convert_type(w, jnp.int32)


def _edge_mm(ea_t, wea, bea, web, beb):
    nb = 25
    eb = N_EDGES // nb
    wspec = pl.BlockSpec((HDIM, EDIM), lambda i: (0, 0))
    bspec = pl.BlockSpec((1, HDIM), lambda i: (0, 0))
    return pl.pallas_call(
        _edge_mm_body,
        grid=(nb,),
        in_specs=[pl.BlockSpec((EDIM, eb), lambda i: (0, i)),
                  wspec, bspec, wspec, bspec],
        out_specs=pl.BlockSpec((eb, HDIM), lambda i: (i, 0)),
        out_shape=jax.ShapeDtypeStruct((N_EDGES, HDIM), jnp.int32),
    )(ea_t, wea, bea, web, beb)


_SC_MESH = plsc.VectorSubcoreMesh(
    core_axis_name="c", subcore_axis_name="s", num_cores=NC, num_subcores=NS)


@functools.partial(
    pl.kernel,
    out_type=jax.ShapeDtypeStruct((NC, N_NODES, DIM), jnp.float32),
    mesh=_SC_MESH,
    compiler_params=pltpu.CompilerParams(needs_layout_passes=False),
    scratch_types=[
        pltpu.VMEM((CB,), jnp.int32),
        pltpu.VMEM((CB,), jnp.int32),
        pltpu.VMEM((CB,), jnp.int32),
        pltpu.VMEM((CB,), jnp.int32),
        pltpu.VMEM((CB,), jnp.int32),
        pltpu.VMEM((CB,), jnp.int32),
        pltpu.VMEM((CB, DIM), jnp.float32),
        pltpu.VMEM((CB, DIM), jnp.float32),
        pltpu.VMEM((CB, DIM), jnp.float32),
        pltpu.VMEM((CB, DIM), jnp.float32),
        pltpu.VMEM((CB, HDIM), jnp.int32),
        pltpu.VMEM((CB, HDIM), jnp.int32),
        pltpu.VMEM((CB, DIM), jnp.float32),
        pltpu.VMEM((CB, DIM), jnp.float32),
        pltpu.VMEM((LANES,), jnp.float32),
        pltpu.VMEM_SHARED((N_NODES, DIM), jnp.float32),
        pltpu.SemaphoreType.DMA,
        pltpu.SemaphoreType.DMA,
        pltpu.SemaphoreType.DMA,
        pltpu.SemaphoreType.DMA,
        pltpu.SemaphoreType.DMA,
        pltpu.SemaphoreType.DMA,
        pltpu.SemaphoreType.DMA,
        pltpu.SemaphoreType.DMA,
    ],
)
def _edge_scatter(hs_hbm, hd_hbm, ep_hbm, src_hbm, dst_hbm, a_hbm, out_hbm,
                  si0, si1, dg0, dg1, ds0, ds1,
                  ba0, ba1, bb0, bb1, bc0, bc1, bz0, bz1,
                  a_v, acc, semg0, semg1, semsc0, semsc1,
                  semi0, semi1, semd0, semd1):
    c = lax.axis_index("c")
    s = lax.axis_index("s")
    wid = c * NS + s
    sibuf = (si0, si1)
    dgbuf = (dg0, dg1)
    dsbuf = (ds0, ds1)
    buf_a = (ba0, ba1)
    buf_b = (bb0, bb1)
    buf_c = (bc0, bc1)
    buf_z = (bz0, bz1)
    semg = (semg0, semg1)
    semsc = (semsc0, semsc1)
    semi = (semi0, semi1)
    semd = (semd0, semd1)

    zero = jnp.zeros((LANES,), jnp.float32)

    def _zrow(e, _):
        for j in range(DIM // LANES):
            bz0[e, pl.ds(j * LANES, LANES)] = zero
        return 0

    lax.fori_loop(0, CB, _zrow, 0)
    base_row = s * STRIP
    for k in range(STRIP // CB):
        pltpu.sync_copy(bz0, acc.at[pl.ds(base_row + k * CB, CB)])
    rem = STRIP % CB
    if rem:
        pltpu.sync_copy(bz0.at[pl.ds(0, rem)],
                        acc.at[pl.ds(base_row + (STRIP // CB) * CB, rem)])

    @pl.when(s == NS - 1)
    def _():
        pltpu.sync_copy(bz0.at[pl.ds(0, TAIL)],
                        acc.at[pl.ds(NS * STRIP, TAIL)])

    pltpu.sync_copy(a_hbm, a_v)
    a = a_v[...]
    himask = jnp.full((LANES,), -65536, jnp.int32)
    plsc.subcore_barrier()

    ebase = wid * EPW

    def _issue_idx(ci, r):
        isl = pl.ds(ebase + ci * CB, CB)
        pltpu.async_copy(src_hbm.at[isl], sibuf[r], semi[r])
        pltpu.async_copy(dst_hbm.at[isl], dgbuf[r], semi[r])

    def _wait_idx(r):
        pltpu.make_async_copy(src_hbm.at[pl.ds(0, CB)], sibuf[r],
                              semi[r]).wait()
        pltpu.make_async_copy(dst_hbm.at[pl.ds(0, CB)], dgbuf[r],
                              semi[r]).wait()

    def _issue_dss(ci, r):
        pltpu.async_copy(dst_hbm.at[pl.ds(ebase + ci * CB, CB)], dsbuf[r],
                         semd[r])

    def _wait_dss(r):
        pltpu.make_async_copy(dst_hbm.at[pl.ds(0, CB)], dsbuf[r],
                              semd[r]).wait()

    def _issue_gathers(ci, r):
        pltpu.async_copy(hs_hbm.at[sibuf[r]], buf_a[r], semg[r])
        pltpu.async_copy(hd_hbm.at[dgbuf[r]], buf_b[r], semg[r])
        pltpu.async_copy(ep_hbm.at[pl.ds(ebase + ci * CB, CB)], buf_c[r],
                         semg[r])

    def _wait_gathers(r):
        pltpu.make_async_copy(hs_hbm.at[sibuf[r]], buf_a[r], semg[r]).wait()
        pltpu.make_async_copy(hd_hbm.at[dgbuf[r]], buf_b[r], semg[r]).wait()
        pltpu.make_async_copy(ep_hbm.at[pl.ds(0, CB)], buf_c[r],
                              semg[r]).wait()

    def _compute(r):
        ra, rb, rc, rz = buf_a[r], buf_b[r], buf_c[r], buf_z[r]

        def _erow(e, _):
            for g in range(GROUPS):
                wv = rc[e, pl.ds(g * LANES, LANES)]
                clo = plsc.bitcast(lax.shift_left(wv, 16), jnp.float32)
                chi = plsc.bitcast(lax.bitwise_and(wv, himask), jnp.float32)
                slo = pl.ds(g * 32, LANES)
                shi = pl.ds(g * 32 + LANES, LANES)
                vlo = ra[e, slo] + rb[e, slo] + clo
                vhi = ra[e, shi] + rb[e, shi] + chi
                rz[e, slo] = (jnp.maximum(vlo, 0.0)
                              + a * jnp.minimum(vlo, 0.0))
                rz[e, shi] = (jnp.maximum(vhi, 0.0)
                              + a * jnp.minimum(vhi, 0.0))
            return 0

        lax.fori_loop(0, CB, _erow, 0)

    def _issue_scatter(r):
        pltpu.async_copy(buf_z[r], acc.at[dsbuf[r]], semsc[r], add=True)

    def _wait_scatter(r):
        pltpu.make_async_copy(buf_z[r], acc.at[dsbuf[r]], semsc[r]).wait()

    _issue_idx(0, 0)
    _wait_idx(0)
    _issue_gathers(0, 0)
    _issue_idx(1, 1)
    _wait_idx(1)
    _issue_gathers(1, 1)
    _wait_gathers(0)
    _issue_idx(2, 0)
    _issue_dss(0, 0)
    _compute(0)
    _wait_dss(0)
    _issue_scatter(0)
    _wait_idx(0)
    _issue_gathers(2, 0)
    _wait_gathers(1)
    _issue_idx(3, 1)
    _issue_dss(1, 1)
    _compute(1)
    _wait_dss(1)
    _issue_scatter(1)

    def _half(ci, p):
        q = 1 - p
        _wait_idx(q)
        _issue_gathers(ci + 1, q)
        _wait_gathers(p)
        _issue_idx(ci + 2, p)
        _wait_scatter(p)
        _issue_dss(ci, p)
        _compute(p)
        _wait_dss(p)
        _issue_scatter(p)

    def _body(k, _):
        _half(2 * k, 0)
        _half(2 * k + 1, 1)
        return 0

    lax.fori_loop(1, NCHUNK // 2 - 1, _body, 0)

    _wait_idx(1)
    _issue_gathers(NCHUNK - 1, 1)
    _wait_gathers(0)
    _wait_scatter(0)
    _issue_dss(NCHUNK - 2, 0)
    _compute(0)
    _wait_dss(0)
    _issue_scatter(0)
    _wait_gathers(1)
    _wait_scatter(1)
    _issue_dss(NCHUNK - 1, 1)
    _compute(1)
    _wait_dss(1)
    _issue_scatter(1)
    _wait_scatter(0)
    _wait_scatter(1)

    plsc.subcore_barrier()
    pltpu.sync_copy(acc.at[pl.ds(base_row, STRIP)],
                    out_hbm.at[c].at[pl.ds(base_row, STRIP)])

    @pl.when(s == NS - 1)
    def _():
        pltpu.sync_copy(acc.at[pl.ds(NS * STRIP, TAIL)],
                        out_hbm.at[c].at[pl.ds(NS * STRIP, TAIL)])


def _bn_stats_body(p_ref, sum_ref, sq_ref):
    i = pl.program_id(0)
    h = p_ref[0] + p_ref[1]
    s = jnp.sum(h, axis=0, keepdims=True)
    q = jnp.sum(h * h, axis=0, keepdims=True)

    @pl.when(i == 0)
    def _():
        sum_ref[...] = s
        sq_ref[...] = q

    @pl.when(i > 0)
    def _():
        sum_ref[...] += s
        sq_ref[...] += q


def _bn_stats(partial):
    nb = 10
    rb = N_NODES // nb
    return pl.pallas_call(
        _bn_stats_body,
        grid=(nb,),
        in_specs=[pl.BlockSpec((NC, rb, DIM), lambda i: (0, i, 0))],
        out_specs=[
            pl.BlockSpec((1, DIM), lambda i: (0, 0)),
            pl.BlockSpec((1, DIM), lambda i: (0, 0)),
        ],
        out_shape=[
            jax.ShapeDtypeStruct((1, DIM), jnp.float32),
            jax.ShapeDtypeStruct((1, DIM), jnp.float32),
        ],
    )(partial)


def _bn_apply_body(p_ref, sum_ref, sq_ref, g_ref, b_ref, a_ref, out_ref):
    inv_n = 1.0 / N_NODES
    mean = sum_ref[...] * inv_n
    var = sq_ref[...] * inv_n - mean * mean
    inv = lax.rsqrt(var + 1e-5)
    scale = inv * g_ref[...]
    shift = b_ref[...] - mean * scale
    h = p_ref[0] + p_ref[1]
    y = h * scale + shift
    a = a_ref[0]
    out_ref[...] = jnp.maximum(y, 0.0) + a * jnp.minimum(y, 0.0)


def _bn_apply(partial, sums, sqs, gamma, beta, a_arr):
    nb = 10
    rb = N_NODES // nb
    return pl.pallas_call(
        _bn_apply_body,
        grid=(nb,),
        in_specs=[
            pl.BlockSpec((NC, rb, DIM), lambda i: (0, i, 0)),
            pl.BlockSpec((1, DIM), lambda i: (0, 0)),
            pl.BlockSpec((1, DIM), lambda i: (0, 0)),
            pl.BlockSpec((1, DIM), lambda i: (0, 0)),
            pl.BlockSpec((1, DIM), lambda i: (0, 0)),
            pl.BlockSpec(memory_space=pltpu.SMEM),
        ],
        out_specs=pl.BlockSpec((rb, DIM), lambda i: (i, 0)),
        out_shape=jax.ShapeDtypeStruct((N_NODES, DIM), jnp.float32),
    )(partial, sums, sqs, gamma.reshape(1, DIM), beta.reshape(1, DIM), a_arr)


def kernel(x, edge_index, edge_attr, W_src, b_src, W_dst, b_dst,
           W_edge, b_edge, prelu_a, bn_gamma, bn_beta):
    ia = jnp.asarray(_IDX_A, jnp.int32)
    ib = jnp.asarray(_IDX_B, jnp.int32)
    hs, hd = _node_mm(x, W_src, b_src, W_dst, b_dst)
    ep = _edge_mm(
        edge_attr.T,
        W_edge[ia], b_edge[ia].reshape(1, HDIM),
        W_edge[ib], b_edge[ib].reshape(1, HDIM))
    src = edge_index[0]
    dst = edge_index[1]
    a_vec = jnp.full((LANES,), prelu_a, jnp.float32)
    partial = _edge_scatter(hs, hd, ep, src, dst, a_vec)
    sums, sqs = _bn_stats(partial)
    a_arr = jnp.full((1,), prelu_a, jnp.float32)
    return _bn_apply(partial, sums, sqs, bn_gamma, bn_beta, a_arr)


# speedup vs baseline: 1.4917x; 1.0156x over previous
import functools

import jax
import jax.numpy as jnp
from jax import lax
from jax.experimental import pallas as pl
from jax.experimental.pallas import tpu as pltpu
from jax.experimental.pallas import tpu_sc as plsc

N_NODES = 10000
N_EDGES = 320000
DIM = 128
HDIM = DIM // 2
EDIM = 17

NC = 2
NS = 16
NW = NC * NS
EPW = N_EDGES // NW
CB = 40
NCHUNK = EPW // CB
LANES = 16
STRIP = 624
TAIL = N_NODES - NS * STRIP
GROUPS = DIM // 32

_IDX_A = tuple(g * 32 + w for g in range(GROUPS) for w in range(LANES))
_IDX_B = tuple(i + LANES for i in _IDX_A)


def _node_mm_body(x_ref, ws_ref, bs_ref, wd_ref, bd_ref, hs_ref, hd_ref):
    xb = x_ref[...]
    dn = (((1,), (1,)), ((), ()))
    hs_ref[...] = lax.dot_general(
        xb, ws_ref[...], dn, preferred_element_type=jnp.float32) + bs_ref[...]
    hd_ref[...] = lax.dot_general(
        xb, wd_ref[...], dn, preferred_element_type=jnp.float32) + bd_ref[...]


def _node_mm(x, W_src, b_src, W_dst, b_dst):
    nb = 10
    rb = N_NODES // nb
    return pl.pallas_call(
        _node_mm_body,
        grid=(nb,),
        in_specs=[
            pl.BlockSpec((rb, DIM), lambda i: (i, 0)),
            pl.BlockSpec((DIM, DIM), lambda i: (0, 0)),
            pl.BlockSpec((1, DIM), lambda i: (0, 0)),
            pl.BlockSpec((DIM, DIM), lambda i: (0, 0)),
            pl.BlockSpec((1, DIM), lambda i: (0, 0)),
        ],
        out_specs=[
            pl.BlockSpec((rb, DIM), lambda i: (i, 0)),
            pl.BlockSpec((rb, DIM), lambda i: (i, 0)),
        ],
        out_shape=[
            jax.ShapeDtypeStruct((N_NODES, DIM), jnp.float32),
            jax.ShapeDtypeStruct((N_NODES, DIM), jnp.float32),
        ],
    )(x, W_src, b_src.reshape(1, DIM), W_dst, b_dst.reshape(1, DIM))


def _edge_mm_body(eat_ref, wea_ref, bea_ref, web_ref, beb_ref, ep_ref):
    eat = eat_ref[...]
    dn = (((0,), (1,)), ((), ()))

    def mm(w_ref, b_ref):
        return lax.dot_general(
            eat, w_ref[...], dn,
            preferred_element_type=jnp.float32) + b_ref[...]

    ya = mm(wea_ref, bea_ref)
    yb = mm(web_ref, beb_ref)
    la = lax.bitcast_convert_type(ya.astype(jnp.bfloat16), jnp.uint16)
    lb = lax.bitcast_convert_type(yb.astype(jnp.bfloat16), jnp.uint16)
    w = la.astype(jnp.uint32) | (lb.astype(jnp.uint32) << 16)
    ep_ref[...] = lax.bitcast_convert_type(w, jnp.int32)


def _edge_mm(ea_t, wea, bea, web, beb):
    nb = 10
    eb = N_EDGES // nb
    wspec = pl.BlockSpec((HDIM, EDIM), lambda i: (0, 0))
    bspec = pl.BlockSpec((1, HDIM), lambda i: (0, 0))
    return pl.pallas_call(
        _edge_mm_body,
        grid=(nb,),
        in_specs=[pl.BlockSpec((EDIM, eb), lambda i: (0, i)),
                  wspec, bspec, wspec, bspec],
        out_specs=pl.BlockSpec((eb, HDIM), lambda i: (i, 0)),
        out_shape=jax.ShapeDtypeStruct((N_EDGES, HDIM), jnp.int32),
    )(ea_t, wea, bea, web, beb)


_SC_MESH = plsc.VectorSubcoreMesh(
    core_axis_name="c", subcore_axis_name="s", num_cores=NC, num_subcores=NS)


@functools.partial(
    pl.kernel,
    out_type=jax.ShapeDtypeStruct((NC, N_NODES, DIM), jnp.float32),
    mesh=_SC_MESH,
    compiler_params=pltpu.CompilerParams(needs_layout_passes=False),
    scratch_types=[
        pltpu.VMEM((CB,), jnp.int32),
        pltpu.VMEM((CB,), jnp.int32),
        pltpu.VMEM((CB,), jnp.int32),
        pltpu.VMEM((CB,), jnp.int32),
        pltpu.VMEM((CB,), jnp.int32),
        pltpu.VMEM((CB,), jnp.int32),
        pltpu.VMEM((CB, DIM), jnp.float32),
        pltpu.VMEM((CB, DIM), jnp.float32),
        pltpu.VMEM((CB, DIM), jnp.float32),
        pltpu.VMEM((CB, DIM), jnp.float32),
        pltpu.VMEM((CB, HDIM), jnp.int32),
        pltpu.VMEM((CB, HDIM), jnp.int32),
        pltpu.VMEM((CB, DIM), jnp.float32),
        pltpu.VMEM((CB, DIM), jnp.float32),
        pltpu.VMEM((LANES,), jnp.float32),
        pltpu.VMEM_SHARED((N_NODES, DIM), jnp.float32),
        pltpu.SemaphoreType.DMA,
        pltpu.SemaphoreType.DMA,
        pltpu.SemaphoreType.DMA,
        pltpu.SemaphoreType.DMA,
        pltpu.SemaphoreType.DMA,
        pltpu.SemaphoreType.DMA,
        pltpu.SemaphoreType.DMA,
        pltpu.SemaphoreType.DMA,
    ],
)
def _edge_scatter(hs_hbm, hd_hbm, ep_hbm, src_hbm, dst_hbm, a_hbm, out_hbm,
                  si0, si1, dg0, dg1, ds0, ds1,
                  ba0, ba1, bb0, bb1, bc0, bc1, bz0, bz1,
                  a_v, acc, semg0, semg1, semsc0, semsc1,
                  semi0, semi1, semd0, semd1):
    c = lax.axis_index("c")
    s = lax.axis_index("s")
    wid = c * NS + s
    sibuf = (si0, si1)
    dgbuf = (dg0, dg1)
    dsbuf = (ds0, ds1)
    buf_a = (ba0, ba1)
    buf_b = (bb0, bb1)
    buf_c = (bc0, bc1)
    buf_z = (bz0, bz1)
    semg = (semg0, semg1)
    semsc = (semsc0, semsc1)
    semi = (semi0, semi1)
    semd = (semd0, semd1)

    zero = jnp.zeros((LANES,), jnp.float32)

    def _zrow(e, _):
        for j in range(DIM // LANES):
            bz0[e, pl.ds(j * LANES, LANES)] = zero
        return 0

    lax.fori_loop(0, CB, _zrow, 0)
    base_row = s * STRIP
    for k in range(STRIP // CB):
        pltpu.sync_copy(bz0, acc.at[pl.ds(base_row + k * CB, CB)])
    rem = STRIP % CB
    if rem:
        pltpu.sync_copy(bz0.at[pl.ds(0, rem)],
                        acc.at[pl.ds(base_row + (STRIP // CB) * CB, rem)])

    @pl.when(s == NS - 1)
    def _():
        pltpu.sync_copy(bz0.at[pl.ds(0, TAIL)],
                        acc.at[pl.ds(NS * STRIP, TAIL)])

    pltpu.sync_copy(a_hbm, a_v)
    a = a_v[...]
    himask = jnp.full((LANES,), -65536, jnp.int32)
    plsc.subcore_barrier()

    ebase = wid * EPW

    def _issue_idx(ci, r):
        isl = pl.ds(ebase + ci * CB, CB)
        pltpu.async_copy(src_hbm.at[isl], sibuf[r], semi[r])
        pltpu.async_copy(dst_hbm.at[isl], dgbuf[r], semi[r])

    def _wait_idx(r):
        pltpu.make_async_copy(src_hbm.at[pl.ds(0, CB)], sibuf[r],
                              semi[r]).wait()
        pltpu.make_async_copy(dst_hbm.at[pl.ds(0, CB)], dgbuf[r],
                              semi[r]).wait()

    def _issue_dss(ci, r):
        pltpu.async_copy(dst_hbm.at[pl.ds(ebase + ci * CB, CB)], dsbuf[r],
                         semd[r])

    def _wait_dss(r):
        pltpu.make_async_copy(dst_hbm.at[pl.ds(0, CB)], dsbuf[r],
                              semd[r]).wait()

    def _issue_gathers(ci, r):
        pltpu.async_copy(hs_hbm.at[sibuf[r]], buf_a[r], semg[r])
        pltpu.async_copy(hd_hbm.at[dgbuf[r]], buf_b[r], semg[r])
        pltpu.async_copy(ep_hbm.at[pl.ds(ebase + ci * CB, CB)], buf_c[r],
                         semg[r])

    def _wait_gathers(r):
        pltpu.make_async_copy(hs_hbm.at[sibuf[r]], buf_a[r], semg[r]).wait()
        pltpu.make_async_copy(hd_hbm.at[dgbuf[r]], buf_b[r], semg[r]).wait()
        pltpu.make_async_copy(ep_hbm.at[pl.ds(0, CB)], buf_c[r],
                              semg[r]).wait()

    def _compute(r):
        ra, rb, rc, rz = buf_a[r], buf_b[r], buf_c[r], buf_z[r]

        def _erow(e, _):
            for g in range(GROUPS):
                wv = rc[e, pl.ds(g * LANES, LANES)]
                clo = plsc.bitcast(lax.shift_left(wv, 16), jnp.float32)
                chi = plsc.bitcast(lax.bitwise_and(wv, himask), jnp.float32)
                slo = pl.ds(g * 32, LANES)
                shi = pl.ds(g * 32 + LANES, LANES)
                vlo = ra[e, slo] + rb[e, slo] + clo
                vhi = ra[e, shi] + rb[e, shi] + chi
                rz[e, slo] = (jnp.maximum(vlo, 0.0)
                              + a * jnp.minimum(vlo, 0.0))
                rz[e, shi] = (jnp.maximum(vhi, 0.0)
                              + a * jnp.minimum(vhi, 0.0))
            return 0

        lax.fori_loop(0, CB, _erow, 0)

    def _issue_scatter(r):
        pltpu.async_copy(buf_z[r], acc.at[dsbuf[r]], semsc[r], add=True)

    def _wait_scatter(r):
        pltpu.make_async_copy(buf_z[r], acc.at[dsbuf[r]], semsc[r]).wait()

    _issue_idx(0, 0)
    _wait_idx(0)
    _issue_gathers(0, 0)
    _issue_idx(1, 1)
    _wait_idx(1)
    _issue_gathers(1, 1)
    _wait_gathers(0)
    _issue_idx(2, 0)
    _issue_dss(0, 0)
    _compute(0)
    _wait_dss(0)
    _issue_scatter(0)
    _wait_idx(0)
    _issue_gathers(2, 0)
    _wait_gathers(1)
    _issue_idx(3, 1)
    _issue_dss(1, 1)
    _compute(1)
    _wait_dss(1)
    _issue_scatter(1)

    def _half(ci, p):
        q = 1 - p
        _wait_idx(q)
        _issue_gathers(ci + 1, q)
        _wait_gathers(p)
        _issue_idx(ci + 2, p)
        _wait_scatter(p)
        _issue_dss(ci, p)
        _compute(p)
        _wait_dss(p)
        _issue_scatter(p)

    def _body(k, _):
        _half(2 * k, 0)
        _half(2 * k + 1, 1)
        return 0

    lax.fori_loop(1, NCHUNK // 2 - 1, _body, 0)

    _wait_idx(1)
    _issue_gathers(NCHUNK - 1, 1)
    _wait_gathers(0)
    _wait_scatter(0)
    _issue_dss(NCHUNK - 2, 0)
    _compute(0)
    _wait_dss(0)
    _issue_scatter(0)
    _wait_gathers(1)
    _wait_scatter(1)
    _issue_dss(NCHUNK - 1, 1)
    _compute(1)
    _wait_dss(1)
    _issue_scatter(1)
    _wait_scatter(0)
    _wait_scatter(1)

    plsc.subcore_barrier()
    pltpu.sync_copy(acc.at[pl.ds(base_row, STRIP)],
                    out_hbm.at[c].at[pl.ds(base_row, STRIP)])

    @pl.when(s == NS - 1)
    def _():
        pltpu.sync_copy(acc.at[pl.ds(NS * STRIP, TAIL)],
                        out_hbm.at[c].at[pl.ds(NS * STRIP, TAIL)])


def _bn_stats_body(p_ref, sum_ref, sq_ref):
    i = pl.program_id(0)
    h = p_ref[0] + p_ref[1]
    s = jnp.sum(h, axis=0, keepdims=True)
    q = jnp.sum(h * h, axis=0, keepdims=True)

    @pl.when(i == 0)
    def _():
        sum_ref[...] = s
        sq_ref[...] = q

    @pl.when(i > 0)
    def _():
        sum_ref[...] += s
        sq_ref[...] += q


def _bn_stats(partial):
    nb = 10
    rb = N_NODES // nb
    return pl.pallas_call(
        _bn_stats_body,
        grid=(nb,),
        in_specs=[pl.BlockSpec((NC, rb, DIM), lambda i: (0, i, 0))],
        out_specs=[
            pl.BlockSpec((1, DIM), lambda i: (0, 0)),
            pl.BlockSpec((1, DIM), lambda i: (0, 0)),
        ],
        out_shape=[
            jax.ShapeDtypeStruct((1, DIM), jnp.float32),
            jax.ShapeDtypeStruct((1, DIM), jnp.float32),
        ],
    )(partial)


def _bn_apply_body(p_ref, sum_ref, sq_ref, g_ref, b_ref, a_ref, out_ref):
    inv_n = 1.0 / N_NODES
    mean = sum_ref[...] * inv_n
    var = sq_ref[...] * inv_n - mean * mean
    inv = lax.rsqrt(var + 1e-5)
    scale = inv * g_ref[...]
    shift = b_ref[...] - mean * scale
    h = p_ref[0] + p_ref[1]
    y = h * scale + shift
    a = a_ref[0]
    out_ref[...] = jnp.maximum(y, 0.0) + a * jnp.minimum(y, 0.0)


def _bn_apply(partial, sums, sqs, gamma, beta, a_arr):
    nb = 10
    rb = N_NODES // nb
    return pl.pallas_call(
        _bn_apply_body,
        grid=(nb,),
        in_specs=[
            pl.BlockSpec((NC, rb, DIM), lambda i: (0, i, 0)),
            pl.BlockSpec((1, DIM), lambda i: (0, 0)),
            pl.BlockSpec((1, DIM), lambda i: (0, 0)),
            pl.BlockSpec((1, DIM), lambda i: (0, 0)),
            pl.BlockSpec((1, DIM), lambda i: (0, 0)),
            pl.BlockSpec(memory_space=pltpu.SMEM),
        ],
        out_specs=pl.BlockSpec((rb, DIM), lambda i: (i, 0)),
        out_shape=jax.ShapeDtypeStruct((N_NODES, DIM), jnp.float32),
    )(partial, sums, sqs, gamma.reshape(1, DIM), beta.reshape(1, DIM), a_arr)


def kernel(x, edge_index, edge_attr, W_src, b_src, W_dst, b_dst,
           W_edge, b_edge, prelu_a, bn_gamma, bn_beta):
    ia = jnp.asarray(_IDX_A, jnp.int32)
    ib = jnp.asarray(_IDX_B, jnp.int32)
    hs, hd = _node_mm(x, W_src, b_src, W_dst, b_dst)
    ep = _edge_mm(
        edge_attr.T,
        W_edge[ia], b_edge[ia].reshape(1, HDIM),
        W_edge[ib], b_edge[ib].reshape(1, HDIM))
    src = edge_index[0]
    dst = edge_index[1]
    a_vec = jnp.full((LANES,), prelu_a, jnp.float32)
    partial = _edge_scatter(hs, hd, ep, src, dst, a_vec)
    sums, sqs = _bn_stats(partial)
    a_arr = jnp.full((1,), prelu_a, jnp.float32)
    return _bn_apply(partial, sums, sqs, bn_gamma, bn_beta, a_arr)
